# Initial kernel scaffold; baseline (speedup 1.0000x reference)
import jax
import jax.numpy as jnp
from jax import lax
from jax.experimental import pallas as pl
from jax.experimental.pallas import tpu as pltpu, tpu_sc as plsc

EMB = 64
NC, NS = 2, 16
NW = NC * NS
IDXW = 128
CH_IR = 8
CH = CH_IR * IDXW


def _build(n_total):
    rows_per_w = n_total // NW
    ir_per_w = rows_per_w // IDXW
    n_chunks = ir_per_w // CH_IR
    mesh = plsc.VectorSubcoreMesh(core_axis_name="c", subcore_axis_name="s")

    def body(idx_hbm, table_hbm, out_hbm, idx_v, rows_v, sem):
        wid = lax.axis_index("s") * NC + lax.axis_index("c")
        ir_base = wid * ir_per_w

        def chunk(ci, carry):
            ir0 = ir_base + ci * CH_IR
            pltpu.sync_copy(idx_hbm.at[pl.ds(ir0, CH_IR)], idx_v)
            copies = []
            for j in range(CH_IR):
                copies.append(
                    pltpu.async_copy(
                        table_hbm.at[idx_v.at[j]],
                        rows_v.at[pl.ds(j * IDXW, IDXW)],
                        sem,
                    )
                )
            for cp in copies:
                cp.wait()
            out0 = pl.multiple_of(ir0 * IDXW, CH)
            pltpu.sync_copy(rows_v, out_hbm.at[pl.ds(out0, CH)])
            return carry

        lax.fori_loop(0, n_chunks, chunk, 0)

    return pl.kernel(
        body,
        out_type=jax.ShapeDtypeStruct((n_total, EMB), jnp.float32),
        mesh=mesh,
        scratch_types=[
            pltpu.VMEM((CH_IR, IDXW), jnp.int32),
            pltpu.VMEM((CH, EMB), jnp.float32),
            pltpu.SemaphoreType.DMA,
        ],
    )


def kernel(x, table):
    B, L = x.shape
    n = B * L
    xf = x.reshape(n // IDXW, IDXW).astype(jnp.int32)
    out = _build(n)(xf, table)
    return out.reshape(B, L, EMB)


# speedup vs baseline: 1.8464x; 1.8464x over previous
import jax
import jax.numpy as jnp
from jax import lax
from jax.experimental import pallas as pl
from jax.experimental.pallas import tpu as pltpu, tpu_sc as plsc

EMB = 64
NC, NS = 2, 16
NW = NC * NS
IDXW = 128
CH_IR = 8
CH = CH_IR * IDXW


def _build(n_total):
    rows_per_w = n_total // NW
    ir_per_w = rows_per_w // IDXW
    n_chunks = ir_per_w // CH_IR
    mesh = plsc.VectorSubcoreMesh(core_axis_name="c", subcore_axis_name="s")

    def body(idx_hbm, table_hbm, out_hbm, idx_v, rows_v, sem):
        wid = lax.axis_index("s") * NC + lax.axis_index("c")
        ir_base = wid * ir_per_w

        def chunk(ci, carry):
            ir0 = ir_base + ci * CH_IR
            pltpu.sync_copy(idx_hbm.at[pl.ds(ir0, CH_IR)], idx_v)
            copies = []
            for j in range(CH_IR):
                copies.append(
                    pltpu.async_copy(
                        table_hbm.at[idx_v.at[j]],
                        rows_v.at[pl.ds(j * IDXW, IDXW)],
                        sem,
                    )
                )
            for cp in copies:
                cp.wait()
            out0 = pl.multiple_of(ir0 * IDXW, CH)
            pltpu.sync_copy(rows_v, out_hbm.at[pl.ds(out0, CH)])
            return carry

        lax.fori_loop(0, n_chunks, chunk, 0)

    return pl.kernel(
        body,
        out_type=jax.ShapeDtypeStruct((n_total, EMB), jnp.float32),
        mesh=mesh,
        scratch_types=[
            pltpu.VMEM((CH_IR, IDXW), jnp.int32),
            pltpu.VMEM((CH, EMB), jnp.float32),
            pltpu.SemaphoreType.DMA,
        ],
        compiler_params=pltpu.CompilerParams(use_tc_tiling_on_sc=False),
    )


def kernel(x, table):
    B, L = x.shape
    n = B * L
    xf = x.reshape(n // IDXW, IDXW).astype(jnp.int32)
    out = _build(n)(xf, table)
    return out.reshape(B, L, EMB)


# speedup vs baseline: 1.8663x; 1.0107x over previous
import jax
import jax.numpy as jnp
from jax import lax
from jax.experimental import pallas as pl
from jax.experimental.pallas import tpu as pltpu, tpu_sc as plsc

EMB = 64
NC, NS = 2, 16
NW = NC * NS
IDXW = 128
CH_IR = 2
CH = CH_IR * IDXW
NBUF = 4


def _build(n_total):
    rows_per_w = n_total // NW
    ir_per_w = rows_per_w // IDXW
    n_chunks = rows_per_w // CH
    n_groups = n_chunks // NBUF
    mesh = plsc.VectorSubcoreMesh(core_axis_name="c", subcore_axis_name="s")

    def body(idx_hbm, table_hbm, out_hbm, idx_v, rows_v, *sems):
        sem_g = sems[:NBUF]
        sem_o = sems[NBUF:]
        wid = lax.axis_index("s") * NC + lax.axis_index("c")
        ir_base = wid * ir_per_w
        row_base = wid * rows_per_w

        pltpu.sync_copy(idx_hbm.at[pl.ds(ir_base, ir_per_w)], idx_v)

        def gather_copies(c, b):
            return [
                pltpu.make_async_copy(
                    table_hbm.at[idx_v.at[c * CH_IR + s]],
                    rows_v.at[b].at[pl.ds(s * IDXW, IDXW)],
                    sem_g[b],
                )
                for s in range(CH_IR)
            ]

        def store_copy(c, b):
            return pltpu.make_async_copy(
                rows_v.at[b],
                out_hbm.at[pl.ds(row_base + c * CH, CH)],
                sem_o[b],
            )

        for b in range(NBUF):
            for cp in gather_copies(b, b):
                cp.start()

        def group(g, carry):
            c0 = g * NBUF
            for b in range(NBUF):
                for cp in gather_copies(c0 + b, b):
                    cp.wait()
                store_copy(c0 + b, b).start()

            @pl.when(g < n_groups - 1)
            def _():
                for b in range(NBUF):
                    store_copy(c0 + b, b).wait()
                    for cp in gather_copies(c0 + NBUF + b, b):
                        cp.start()

            return carry

        lax.fori_loop(0, n_groups, group, 0)

        for b in range(NBUF):
            store_copy((n_groups - 1) * NBUF + b, b).wait()

    return pl.kernel(
        body,
        out_type=jax.ShapeDtypeStruct((n_total, EMB), jnp.float32),
        mesh=mesh,
        scratch_types=[
            pltpu.VMEM((ir_per_w, IDXW), jnp.int32),
            pltpu.VMEM((NBUF, CH, EMB), jnp.float32),
        ]
        + [pltpu.SemaphoreType.DMA] * (2 * NBUF),
        compiler_params=pltpu.CompilerParams(use_tc_tiling_on_sc=False),
    )


def kernel(x, table):
    B, L = x.shape
    n = B * L
    xf = x.reshape(n // IDXW, IDXW).astype(jnp.int32)
    out = _build(n)(xf, table)
    return out.reshape(B, L, EMB)


# speedup vs baseline: 1.8665x; 1.0001x over previous
import jax
import jax.numpy as jnp
from jax import lax
from jax.experimental import pallas as pl
from jax.experimental.pallas import tpu as pltpu, tpu_sc as plsc

EMB = 64
NC, NS = 2, 16
NW = NC * NS
NB = 8
NBUF = 4


def _build(n_rows, seq):
    rows_per_w = n_rows // NW
    n_chunks = rows_per_w // NB
    n_groups = n_chunks // NBUF
    mesh = plsc.VectorSubcoreMesh(core_axis_name="c", subcore_axis_name="s")

    def body(idx_hbm, table_hbm, out_hbm, idx_v, rows_v, *sems):
        sem_g = sems[:NBUF]
        sem_o = sems[NBUF:]
        wid = lax.axis_index("s") * NC + lax.axis_index("c")
        r_base = wid * rows_per_w

        pltpu.sync_copy(idx_hbm.at[pl.ds(r_base, rows_per_w)], idx_v)

        def gather_copies(c, b):
            return [
                pltpu.make_async_copy(
                    table_hbm.at[idx_v.at[c * NB + j]],
                    rows_v.at[b].at[j],
                    sem_g[b],
                )
                for j in range(NB)
            ]

        def store_copy(c, b):
            return pltpu.make_async_copy(
                rows_v.at[b],
                out_hbm.at[pl.ds(r_base + c * NB, NB)],
                sem_o[b],
            )

        for b in range(NBUF):
            for cp in gather_copies(b, b):
                cp.start()

        def group(g, carry):
            c0 = g * NBUF
            for b in range(NBUF):
                for cp in gather_copies(c0 + b, b):
                    cp.wait()
                store_copy(c0 + b, b).start()

            @pl.when(g < n_groups - 1)
            def _():
                for b in range(NBUF):
                    store_copy(c0 + b, b).wait()
                    for cp in gather_copies(c0 + NBUF + b, b):
                        cp.start()

            return carry

        lax.fori_loop(0, n_groups, group, 0)

        for b in range(NBUF):
            store_copy((n_groups - 1) * NBUF + b, b).wait()

    return pl.kernel(
        body,
        out_type=jax.ShapeDtypeStruct((n_rows, seq, EMB), jnp.float32),
        mesh=mesh,
        scratch_types=[
            pltpu.VMEM((rows_per_w, seq), jnp.int32),
            pltpu.VMEM((NBUF, NB, seq, EMB), jnp.float32),
        ]
        + [pltpu.SemaphoreType.DMA] * (2 * NBUF),
        compiler_params=pltpu.CompilerParams(use_tc_tiling_on_sc=False),
    )


def kernel(x, table):
    B, L = x.shape
    return _build(B, L)(x, table)
